# baseline (speedup 1.0000x reference)
import functools

import jax
import jax.numpy as jnp
from jax import lax
from jax.experimental import pallas as pl
from jax.experimental.pallas import tpu as pltpu
from jax.experimental.pallas import tpu_sc as plsc

N_NODES = 10000
NP = 10240
E = 320000
CHUNK = 120
K = 85
KX = 90
EP = 32 * K * CHUNK
DCHUNK = 120
KD = EP // (16 * DCHUNK)
PAD = N_NODES
NS = 16
RPT = NP // NS


@functools.cache
def _sc_degrees():
    mesh = plsc.VectorSubcoreMesh(core_axis_name="c", subcore_axis_name="s")

    @functools.partial(
        pl.kernel,
        out_type=(jax.ShapeDtypeStruct((NP,), jnp.float32),
                  jax.ShapeDtypeStruct((NP,), jnp.float32)),
        mesh=mesh,
        scratch_types=[
            pltpu.VMEM((KD, DCHUNK), jnp.int32),
            pltpu.VMEM((128,), jnp.float32),
            pltpu.VMEM((RPT,), jnp.float32),
            pltpu.VMEM_SHARED((NP,), jnp.float32),
            pltpu.SemaphoreType.DMA,
            pltpu.SemaphoreType.DMA,
        ],
    )
    def deg_kernel(src_hbm, dst_hbm, dego_hbm, degi_hbm,
                   idx_v, ones_v, zeros_v, deg_sp, sem_a, sem_b):
        c = lax.axis_index("c")
        s = lax.axis_index("s")

        @pl.loop(0, 128, step=16)
        def _(j):
            ones_v[pl.ds(j, 16)] = jnp.ones((16,), jnp.float32)

        @pl.loop(0, RPT, step=16)
        def _(j):
            zeros_v[pl.ds(j, 16)] = jnp.zeros((16,), jnp.float32)

        @pl.when(c == 0)
        def _():
            pltpu.sync_copy(src_hbm.at[s], idx_v)

        @pl.when(c == 1)
        def _():
            pltpu.sync_copy(dst_hbm.at[s], idx_v)

        pltpu.sync_copy(zeros_v, deg_sp.at[pl.ds(s * RPT, RPT)])
        plsc.subcore_barrier()

        @pl.loop(0, KD, step=2)
        def _(j):
            ones_c = ones_v.at[pl.ds(0, DCHUNK)]
            pltpu.async_copy(ones_c, deg_sp.at[idx_v.at[j]], sem_a, add=True)
            pltpu.async_copy(ones_c, deg_sp.at[idx_v.at[j + 1]], sem_b,
                             add=True)
            pltpu.make_async_copy(ones_c, deg_sp.at[idx_v.at[j]], sem_a).wait()
            pltpu.make_async_copy(ones_c, deg_sp.at[idx_v.at[j + 1]],
                                  sem_b).wait()

        plsc.subcore_barrier()

        @pl.when((s == 0) & (c == 0))
        def _():
            pltpu.sync_copy(deg_sp, dego_hbm)

        @pl.when((s == 0) & (c == 1))
        def _():
            pltpu.sync_copy(deg_sp, degi_hbm)

    return deg_kernel


@functools.cache
def _sc_aggregate(D):
    mesh = plsc.VectorSubcoreMesh(core_axis_name="c", subcore_axis_name="s")

    @functools.partial(
        pl.kernel,
        out_type=(jax.ShapeDtypeStruct((NP, D), jnp.float32),
                  jax.ShapeDtypeStruct((NP, D), jnp.float32)),
        mesh=mesh,
        scratch_types=[
            pltpu.VMEM((6, 1, CHUNK), jnp.int32),
            pltpu.VMEM((6, 1, CHUNK), jnp.int32),
            pltpu.VMEM((3, CHUNK, D), jnp.float32),
            pltpu.VMEM_SHARED((NP, D), jnp.float32),
        ] + [pltpu.SemaphoreType.DMA] * 12,
    )
    def agg_kernel(h_hbm, src_hbm, dst_hbm, out0_hbm, out1_hbm,
                   src_r, dst_r, rows, agg_sp, *sems):
        c = lax.axis_index("c")
        s = lax.axis_index("s")
        w = c * NS + s
        gsem = sems[0:3]
        ssem = sems[3:6]
        isem = sems[6:12]

        def fire_idx(v, j):
            pltpu.async_copy(src_hbm.at[w, v], src_r.at[j], isem[j])
            pltpu.async_copy(dst_hbm.at[w, v], dst_r.at[j], isem[j])

        def wait_idx(v, j):
            pltpu.make_async_copy(src_hbm.at[w, v], src_r.at[j],
                                  isem[j]).wait()
            pltpu.make_async_copy(dst_hbm.at[w, v], dst_r.at[j],
                                  isem[j]).wait()

        def fire_gather(j, b):
            pltpu.async_copy(h_hbm.at[src_r.at[j, 0]], rows.at[b], gsem[b])

        def wait_gather(j, b):
            pltpu.make_async_copy(h_hbm.at[src_r.at[j, 0]], rows.at[b],
                                  gsem[b]).wait()

        def fire_scatter(j, b):
            pltpu.async_copy(rows.at[b], agg_sp.at[dst_r.at[j, 0]], ssem[b],
                             add=True)

        def wait_scatter(j, b):
            pltpu.make_async_copy(rows.at[b], agg_sp.at[dst_r.at[j, 0]],
                                  ssem[b]).wait()

        @pl.loop(0, CHUNK)
        def _(i):
            @pl.loop(0, D, step=16)
            def _(j):
                rows[0, i, pl.ds(j, 16)] = jnp.zeros((16,), jnp.float32)

        @pl.loop(0, 5)
        def _(r):
            pltpu.sync_copy(rows.at[0],
                            agg_sp.at[pl.ds(s * RPT + r * CHUNK, CHUNK)])

        pltpu.sync_copy(rows.at[0, pl.ds(0, RPT - 5 * CHUNK)],
                        agg_sp.at[pl.ds(s * RPT + 5 * CHUNK,
                                        RPT - 5 * CHUNK)])

        plsc.subcore_barrier()

        def visit(v, b, j_self, with_scatter_wait):
            b2 = (b + 2) % 3
            j2 = (j_self + 2) % 6
            j5 = (j_self + 5) % 6
            wait_gather(j_self, b)
            fire_scatter(j_self, b)
            if with_scatter_wait:
                wait_scatter((j_self + 5) % 6, b2)
                fire_idx(v + 5, j5)
            wait_idx(v + 2, j2)
            fire_gather(j2, b2)

        pltpu.sync_copy(src_hbm.at[w, 0], src_r.at[0])
        pltpu.sync_copy(dst_hbm.at[w, 0], dst_r.at[0])
        pltpu.sync_copy(src_hbm.at[w, 1], src_r.at[1])
        pltpu.sync_copy(dst_hbm.at[w, 1], dst_r.at[1])
        fire_idx(2, 2)
        fire_idx(3, 3)
        fire_idx(4, 4)
        fire_gather(0, 0)
        fire_gather(1, 1)

        wait_gather(0, 0)
        fire_scatter(0, 0)
        fire_idx(5, 5)
        wait_idx(2, 2)
        fire_gather(2, 2)

        @pl.loop(1, K, step=6)
        def _(i):
            for t in range(6):
                visit(i + t, (1 + t) % 3, (1 + t) % 6, True)

        wait_scatter((K - 1) % 6, (K - 1) % 3)
        wait_gather(K % 6, K % 3)
        wait_gather((K + 1) % 6, (K + 1) % 3)
        wait_idx(K + 2, (K + 2) % 6)
        wait_idx(K + 3, (K + 3) % 6)
        wait_idx(K + 4, (K + 4) % 6)

        plsc.subcore_barrier()

        @pl.when(c == 0)
        def _():
            pltpu.sync_copy(agg_sp.at[pl.ds(s * RPT, RPT)],
                            out0_hbm.at[pl.ds(s * RPT, RPT)])

        @pl.when(c == 1)
        def _():
            pltpu.sync_copy(agg_sp.at[pl.ds(s * RPT, RPT)],
                            out1_hbm.at[pl.ds(s * RPT, RPT)])

    return agg_kernel


_BR = 2048


def _norm(deg):
    return lax.rsqrt(jnp.maximum(deg, 1.0))


def _tc_first(x, deg_out, W):
    def body(x_ref, do_ref, w_ref, o_ref):
        xs = x_ref[...] * _norm(do_ref[...])
        o_ref[...] = jnp.dot(xs, w_ref[...], preferred_element_type=jnp.float32)

    return pl.pallas_call(
        body,
        grid=(NP // _BR,),
        in_specs=[
            pl.BlockSpec((_BR, 128), lambda i: (i, 0)),
            pl.BlockSpec((_BR, 1), lambda i: (i, 0)),
            pl.BlockSpec((128, 128), lambda i: (0, 0)),
        ],
        out_specs=pl.BlockSpec((_BR, 128), lambda i: (i, 0)),
        out_shape=jax.ShapeDtypeStruct((NP, 128), jnp.float32),
    )(x, deg_out, W)


def _tc_mid(a0, a1, deg_in, deg_out, b, W, DO):
    def body(a0_ref, a1_ref, di_ref, do_ref, b_ref, w_ref, o_ref):
        a = (a0_ref[...] + a1_ref[...]) * _norm(di_ref[...]) + b_ref[...]
        xs = jax.nn.relu(a) * _norm(do_ref[...])
        o_ref[...] = jnp.dot(xs, w_ref[...], preferred_element_type=jnp.float32)

    return pl.pallas_call(
        body,
        grid=(NP // _BR,),
        in_specs=[
            pl.BlockSpec((_BR, 128), lambda i: (i, 0)),
            pl.BlockSpec((_BR, 128), lambda i: (i, 0)),
            pl.BlockSpec((_BR, 1), lambda i: (i, 0)),
            pl.BlockSpec((_BR, 1), lambda i: (i, 0)),
            pl.BlockSpec((1, 128), lambda i: (0, 0)),
            pl.BlockSpec((128, DO), lambda i: (0, 0)),
        ],
        out_specs=pl.BlockSpec((_BR, DO), lambda i: (i, 0)),
        out_shape=jax.ShapeDtypeStruct((NP, DO), jnp.float32),
    )(a0, a1, deg_in, deg_out, b, W)


def _tc_final(a0, a1, deg_in, b):
    def body(a0_ref, a1_ref, di_ref, b_ref, o_ref):
        a = (a0_ref[...] + a1_ref[...]) * _norm(di_ref[...]) + b_ref[...]
        o_ref[...] = jax.nn.relu(a[:, :40])

    return pl.pallas_call(
        body,
        grid=(10,),
        in_specs=[
            pl.BlockSpec((1000, 128), lambda i: (i, 0)),
            pl.BlockSpec((1000, 128), lambda i: (i, 0)),
            pl.BlockSpec((1000, 1), lambda i: (i, 0)),
            pl.BlockSpec((1, 128), lambda i: (0, 0)),
        ],
        out_specs=pl.BlockSpec((1000, 40), lambda i: (i, 0)),
        out_shape=jax.ShapeDtypeStruct((N_NODES, 40), jnp.float32),
    )(a0, a1, deg_in, b)


def kernel(in_feat, edge_index, W1, b1, W2, b2, W3, b3):
    src = edge_index[0].astype(jnp.int32)
    dst = edge_index[1].astype(jnp.int32)
    pad_idx = PAD + jnp.arange(EP - E, dtype=jnp.int32) % (NP - N_NODES)
    src_flat = jnp.concatenate([src, pad_idx])
    dst_flat = jnp.concatenate([dst, pad_idx])
    src_d = src_flat.reshape(NS, KD, DCHUNK)
    dst_d = dst_flat.reshape(NS, KD, DCHUNK)
    dummy = (PAD + jnp.arange(32 * (KX - K) * CHUNK, dtype=jnp.int32)
             % (NP - N_NODES)).reshape(32, KX - K, CHUNK)
    src_p = jnp.concatenate([src_flat.reshape(32, K, CHUNK), dummy],
                            axis=1).reshape(32, KX, 1, CHUNK)
    dst_p = jnp.concatenate([dst_flat.reshape(32, K, CHUNK), dummy],
                            axis=1).reshape(32, KX, 1, CHUNK)

    x_p = jnp.concatenate(
        [in_feat, jnp.zeros((NP - N_NODES, 128), jnp.float32)])

    deg_out, deg_in = _sc_degrees()(src_d, dst_d)
    deg_out = deg_out.reshape(NP, 1)
    deg_in = deg_in.reshape(NP, 1)

    W3p = jnp.pad(W3, ((0, 0), (0, 128 - 40)))
    b3p = jnp.pad(b3, (0, 128 - 40)).reshape(1, 128)

    h1 = _tc_first(x_p, deg_out, W1)
    a1_0, a1_1 = _sc_aggregate(128)(h1, src_p, dst_p)
    h2 = _tc_mid(a1_0, a1_1, deg_in, deg_out, b1.reshape(1, 128), W2, 128)
    a2_0, a2_1 = _sc_aggregate(128)(h2, src_p, dst_p)
    h3 = _tc_mid(a2_0, a2_1, deg_in, deg_out, b2.reshape(1, 128), W3p, 128)
    a3_0, a3_1 = _sc_aggregate(128)(h3, src_p, dst_p)
    return _tc_final(a3_0, a3_1, deg_in, b3p)


# speedup vs baseline: 1.0019x; 1.0019x over previous
import functools

import jax
import jax.numpy as jnp
from jax import lax
from jax.experimental import pallas as pl
from jax.experimental.pallas import tpu as pltpu
from jax.experimental.pallas import tpu_sc as plsc

N_NODES = 10000
NP = 10240
E = 320000
CHUNK = 112
K = 91
KX = 96
EP = 32 * K * CHUNK
DCHUNK = 112
KD = EP // (16 * DCHUNK)
PAD = N_NODES
NS = 16
RPT = NP // NS


@functools.cache
def _sc_degrees():
    mesh = plsc.VectorSubcoreMesh(core_axis_name="c", subcore_axis_name="s")

    @functools.partial(
        pl.kernel,
        out_type=(jax.ShapeDtypeStruct((NP,), jnp.float32),
                  jax.ShapeDtypeStruct((NP,), jnp.float32)),
        mesh=mesh,
        scratch_types=[
            pltpu.VMEM((KD, DCHUNK), jnp.int32),
            pltpu.VMEM((128,), jnp.float32),
            pltpu.VMEM((RPT,), jnp.float32),
            pltpu.VMEM_SHARED((NP,), jnp.float32),
            pltpu.SemaphoreType.DMA,
            pltpu.SemaphoreType.DMA,
        ],
    )
    def deg_kernel(src_hbm, dst_hbm, dego_hbm, degi_hbm,
                   idx_v, ones_v, zeros_v, deg_sp, sem_a, sem_b):
        c = lax.axis_index("c")
        s = lax.axis_index("s")

        @pl.loop(0, 128, step=16)
        def _(j):
            ones_v[pl.ds(j, 16)] = jnp.ones((16,), jnp.float32)

        @pl.loop(0, RPT, step=16)
        def _(j):
            zeros_v[pl.ds(j, 16)] = jnp.zeros((16,), jnp.float32)

        @pl.when(c == 0)
        def _():
            pltpu.sync_copy(src_hbm.at[s], idx_v)

        @pl.when(c == 1)
        def _():
            pltpu.sync_copy(dst_hbm.at[s], idx_v)

        pltpu.sync_copy(zeros_v, deg_sp.at[pl.ds(s * RPT, RPT)])
        plsc.subcore_barrier()

        @pl.loop(0, KD, step=2)
        def _(j):
            ones_c = ones_v.at[pl.ds(0, DCHUNK)]
            pltpu.async_copy(ones_c, deg_sp.at[idx_v.at[j]], sem_a, add=True)
            pltpu.async_copy(ones_c, deg_sp.at[idx_v.at[j + 1]], sem_b,
                             add=True)
            pltpu.make_async_copy(ones_c, deg_sp.at[idx_v.at[j]], sem_a).wait()
            pltpu.make_async_copy(ones_c, deg_sp.at[idx_v.at[j + 1]],
                                  sem_b).wait()

        plsc.subcore_barrier()

        @pl.when((s == 0) & (c == 0))
        def _():
            pltpu.sync_copy(deg_sp, dego_hbm)

        @pl.when((s == 0) & (c == 1))
        def _():
            pltpu.sync_copy(deg_sp, degi_hbm)

    return deg_kernel


@functools.cache
def _sc_aggregate(D):
    mesh = plsc.VectorSubcoreMesh(core_axis_name="c", subcore_axis_name="s")

    @functools.partial(
        pl.kernel,
        out_type=(jax.ShapeDtypeStruct((NP, D), jnp.float32),
                  jax.ShapeDtypeStruct((NP, D), jnp.float32)),
        mesh=mesh,
        scratch_types=[
            pltpu.VMEM((6, 1, CHUNK), jnp.int32),
            pltpu.VMEM((6, 1, CHUNK), jnp.int32),
            pltpu.VMEM((3, CHUNK, D), jnp.float32),
            pltpu.VMEM_SHARED((NP, D), jnp.float32),
        ] + [pltpu.SemaphoreType.DMA] * 12,
    )
    def agg_kernel(h_hbm, src_hbm, dst_hbm, out0_hbm, out1_hbm,
                   src_r, dst_r, rows, agg_sp, *sems):
        c = lax.axis_index("c")
        s = lax.axis_index("s")
        w = c * NS + s
        gsem = sems[0:3]
        ssem = sems[3:6]
        isem = sems[6:12]

        def fire_idx(v, j):
            pltpu.async_copy(src_hbm.at[w, v], src_r.at[j], isem[j])
            pltpu.async_copy(dst_hbm.at[w, v], dst_r.at[j], isem[j])

        def wait_idx(v, j):
            pltpu.make_async_copy(src_hbm.at[w, v], src_r.at[j],
                                  isem[j]).wait()
            pltpu.make_async_copy(dst_hbm.at[w, v], dst_r.at[j],
                                  isem[j]).wait()

        def fire_gather(j, b):
            pltpu.async_copy(h_hbm.at[src_r.at[j, 0]], rows.at[b], gsem[b])

        def wait_gather(j, b):
            pltpu.make_async_copy(h_hbm.at[src_r.at[j, 0]], rows.at[b],
                                  gsem[b]).wait()

        def fire_scatter(j, b):
            pltpu.async_copy(rows.at[b], agg_sp.at[dst_r.at[j, 0]], ssem[b],
                             add=True)

        def wait_scatter(j, b):
            pltpu.make_async_copy(rows.at[b], agg_sp.at[dst_r.at[j, 0]],
                                  ssem[b]).wait()

        @pl.loop(0, CHUNK)
        def _(i):
            @pl.loop(0, D, step=16)
            def _(j):
                rows[0, i, pl.ds(j, 16)] = jnp.zeros((16,), jnp.float32)

        @pl.loop(0, 5)
        def _(r):
            pltpu.sync_copy(rows.at[0],
                            agg_sp.at[pl.ds(s * RPT + r * CHUNK, CHUNK)])

        pltpu.sync_copy(rows.at[0, pl.ds(0, RPT - 5 * CHUNK)],
                        agg_sp.at[pl.ds(s * RPT + 5 * CHUNK,
                                        RPT - 5 * CHUNK)])

        plsc.subcore_barrier()

        def visit(v, b, j_self, with_scatter_wait):
            b2 = (b + 2) % 3
            j2 = (j_self + 2) % 6
            j5 = (j_self + 5) % 6
            wait_gather(j_self, b)
            fire_scatter(j_self, b)
            if with_scatter_wait:
                wait_scatter((j_self + 5) % 6, b2)
                fire_idx(v + 5, j5)
            wait_idx(v + 2, j2)
            fire_gather(j2, b2)

        pltpu.sync_copy(src_hbm.at[w, 0], src_r.at[0])
        pltpu.sync_copy(dst_hbm.at[w, 0], dst_r.at[0])
        pltpu.sync_copy(src_hbm.at[w, 1], src_r.at[1])
        pltpu.sync_copy(dst_hbm.at[w, 1], dst_r.at[1])
        fire_idx(2, 2)
        fire_idx(3, 3)
        fire_idx(4, 4)
        fire_gather(0, 0)
        fire_gather(1, 1)

        wait_gather(0, 0)
        fire_scatter(0, 0)
        fire_idx(5, 5)
        wait_idx(2, 2)
        fire_gather(2, 2)

        @pl.loop(1, K, step=6)
        def _(i):
            for t in range(6):
                visit(i + t, (1 + t) % 3, (1 + t) % 6, True)

        wait_scatter((K - 1) % 6, (K - 1) % 3)
        wait_gather(K % 6, K % 3)
        wait_gather((K + 1) % 6, (K + 1) % 3)
        wait_idx(K + 2, (K + 2) % 6)
        wait_idx(K + 3, (K + 3) % 6)
        wait_idx(K + 4, (K + 4) % 6)

        plsc.subcore_barrier()

        @pl.when(c == 0)
        def _():
            pltpu.sync_copy(agg_sp.at[pl.ds(s * RPT, RPT)],
                            out0_hbm.at[pl.ds(s * RPT, RPT)])

        @pl.when(c == 1)
        def _():
            pltpu.sync_copy(agg_sp.at[pl.ds(s * RPT, RPT)],
                            out1_hbm.at[pl.ds(s * RPT, RPT)])

    return agg_kernel


_BR = 2048


def _norm(deg):
    return lax.rsqrt(jnp.maximum(deg, 1.0))


def _tc_matmul(x, W):
    def body(x_ref, w_ref, o_ref):
        o_ref[...] = jnp.dot(x_ref[...], w_ref[...],
                             preferred_element_type=jnp.float32)

    return pl.pallas_call(
        body,
        grid=(NP // _BR,),
        in_specs=[
            pl.BlockSpec((_BR, 128), lambda i: (i, 0)),
            pl.BlockSpec((128, 128), lambda i: (0, 0)),
        ],
        out_specs=pl.BlockSpec((_BR, 128), lambda i: (i, 0)),
        out_shape=jax.ShapeDtypeStruct((NP, 128), jnp.float32),
    )(x, W)


def _tc_scale(y, deg_out):
    def body(y_ref, do_ref, o_ref):
        o_ref[...] = y_ref[...] * _norm(do_ref[...])

    return pl.pallas_call(
        body,
        grid=(NP // _BR,),
        in_specs=[
            pl.BlockSpec((_BR, 128), lambda i: (i, 0)),
            pl.BlockSpec((_BR, 1), lambda i: (i, 0)),
        ],
        out_specs=pl.BlockSpec((_BR, 128), lambda i: (i, 0)),
        out_shape=jax.ShapeDtypeStruct((NP, 128), jnp.float32),
    )(y, deg_out)


def _tc_mid(a0, a1, deg_in, deg_out, b, W, DO):
    def body(a0_ref, a1_ref, di_ref, do_ref, b_ref, w_ref, o_ref):
        a = (a0_ref[...] + a1_ref[...]) * _norm(di_ref[...]) + b_ref[...]
        xs = jax.nn.relu(a) * _norm(do_ref[...])
        o_ref[...] = jnp.dot(xs, w_ref[...], preferred_element_type=jnp.float32)

    return pl.pallas_call(
        body,
        grid=(NP // _BR,),
        in_specs=[
            pl.BlockSpec((_BR, 128), lambda i: (i, 0)),
            pl.BlockSpec((_BR, 128), lambda i: (i, 0)),
            pl.BlockSpec((_BR, 1), lambda i: (i, 0)),
            pl.BlockSpec((_BR, 1), lambda i: (i, 0)),
            pl.BlockSpec((1, 128), lambda i: (0, 0)),
            pl.BlockSpec((128, DO), lambda i: (0, 0)),
        ],
        out_specs=pl.BlockSpec((_BR, DO), lambda i: (i, 0)),
        out_shape=jax.ShapeDtypeStruct((NP, DO), jnp.float32),
    )(a0, a1, deg_in, deg_out, b, W)


def _tc_final(a0, a1, deg_in, b):
    def body(a0_ref, a1_ref, di_ref, b_ref, o_ref):
        a = (a0_ref[...] + a1_ref[...]) * _norm(di_ref[...]) + b_ref[...]
        o_ref[...] = jax.nn.relu(a[:, :40])

    return pl.pallas_call(
        body,
        grid=(10,),
        in_specs=[
            pl.BlockSpec((1000, 128), lambda i: (i, 0)),
            pl.BlockSpec((1000, 128), lambda i: (i, 0)),
            pl.BlockSpec((1000, 1), lambda i: (i, 0)),
            pl.BlockSpec((1, 128), lambda i: (0, 0)),
        ],
        out_specs=pl.BlockSpec((1000, 40), lambda i: (i, 0)),
        out_shape=jax.ShapeDtypeStruct((N_NODES, 40), jnp.float32),
    )(a0, a1, deg_in, b)


def kernel(in_feat, edge_index, W1, b1, W2, b2, W3, b3):
    src = edge_index[0].astype(jnp.int32)
    dst = edge_index[1].astype(jnp.int32)
    pad_idx = PAD + jnp.arange(EP - E, dtype=jnp.int32) % (NP - N_NODES)
    src_flat = jnp.concatenate([src, pad_idx])
    dst_flat = jnp.concatenate([dst, pad_idx])
    src_d = src_flat.reshape(NS, KD, DCHUNK)
    dst_d = dst_flat.reshape(NS, KD, DCHUNK)
    dummy = (PAD + jnp.arange(32 * (KX - K) * CHUNK, dtype=jnp.int32)
             % (NP - N_NODES)).reshape(32, KX - K, CHUNK)
    src_p = jnp.concatenate([src_flat.reshape(32, K, CHUNK), dummy],
                            axis=1).reshape(32, KX, 1, CHUNK)
    dst_p = jnp.concatenate([dst_flat.reshape(32, K, CHUNK), dummy],
                            axis=1).reshape(32, KX, 1, CHUNK)

    x_p = jnp.concatenate(
        [in_feat, jnp.zeros((NP - N_NODES, 128), jnp.float32)])

    deg_out, deg_in = _sc_degrees()(src_d, dst_d)
    deg_out = deg_out.reshape(NP, 1)
    deg_in = deg_in.reshape(NP, 1)

    W3p = jnp.pad(W3, ((0, 0), (0, 128 - 40)))
    b3p = jnp.pad(b3, (0, 128 - 40)).reshape(1, 128)

    h1 = _tc_scale(_tc_matmul(x_p, W1), deg_out)
    a1_0, a1_1 = _sc_aggregate(128)(h1, src_p, dst_p)
    h2 = _tc_mid(a1_0, a1_1, deg_in, deg_out, b1.reshape(1, 128), W2, 128)
    a2_0, a2_1 = _sc_aggregate(128)(h2, src_p, dst_p)
    h3 = _tc_mid(a2_0, a2_1, deg_in, deg_out, b2.reshape(1, 128), W3p, 128)
    a3_0, a3_1 = _sc_aggregate(128)(h3, src_p, dst_p)
    return _tc_final(a3_0, a3_1, deg_in, b3p)


# speedup vs baseline: 1.0113x; 1.0094x over previous
import functools

import jax
import jax.numpy as jnp
from jax import lax
from jax.experimental import pallas as pl
from jax.experimental.pallas import tpu as pltpu
from jax.experimental.pallas import tpu_sc as plsc

N_NODES = 10000
NP = 10240
E = 320000
CHUNK = 112
K = 91
KX = 96
EP = 32 * K * CHUNK
DCHUNK = 112
KD = EP // (16 * DCHUNK)
PAD = N_NODES
NS = 16
RPT = NP // NS


@functools.cache
def _sc_degrees():
    mesh = plsc.VectorSubcoreMesh(core_axis_name="c", subcore_axis_name="s")

    @functools.partial(
        pl.kernel,
        out_type=(jax.ShapeDtypeStruct((NP,), jnp.float32),
                  jax.ShapeDtypeStruct((NP,), jnp.float32)),
        mesh=mesh,
        scratch_types=[
            pltpu.VMEM((KD, DCHUNK), jnp.int32),
            pltpu.VMEM((128,), jnp.float32),
            pltpu.VMEM((RPT,), jnp.float32),
            pltpu.VMEM_SHARED((NP,), jnp.float32),
            pltpu.SemaphoreType.DMA,
            pltpu.SemaphoreType.DMA,
        ],
    )
    def deg_kernel(src_hbm, dst_hbm, dego_hbm, degi_hbm,
                   idx_v, ones_v, zeros_v, deg_sp, sem_a, sem_b):
        c = lax.axis_index("c")
        s = lax.axis_index("s")

        @pl.loop(0, 128, step=16)
        def _(j):
            ones_v[pl.ds(j, 16)] = jnp.ones((16,), jnp.float32)

        @pl.loop(0, RPT, step=16)
        def _(j):
            zeros_v[pl.ds(j, 16)] = jnp.zeros((16,), jnp.float32)

        @pl.when(c == 0)
        def _():
            pltpu.sync_copy(src_hbm.at[s], idx_v)

        @pl.when(c == 1)
        def _():
            pltpu.sync_copy(dst_hbm.at[s], idx_v)

        pltpu.sync_copy(zeros_v, deg_sp.at[pl.ds(s * RPT, RPT)])
        plsc.subcore_barrier()

        @pl.loop(0, KD, step=2)
        def _(j):
            ones_c = ones_v.at[pl.ds(0, DCHUNK)]
            pltpu.async_copy(ones_c, deg_sp.at[idx_v.at[j]], sem_a, add=True)
            pltpu.async_copy(ones_c, deg_sp.at[idx_v.at[j + 1]], sem_b,
                             add=True)
            pltpu.make_async_copy(ones_c, deg_sp.at[idx_v.at[j]], sem_a).wait()
            pltpu.make_async_copy(ones_c, deg_sp.at[idx_v.at[j + 1]],
                                  sem_b).wait()

        plsc.subcore_barrier()

        @pl.when((s == 0) & (c == 0))
        def _():
            pltpu.sync_copy(deg_sp, dego_hbm)

        @pl.when((s == 0) & (c == 1))
        def _():
            pltpu.sync_copy(deg_sp, degi_hbm)

    return deg_kernel


@functools.cache
def _sc_aggregate(D):
    mesh = plsc.VectorSubcoreMesh(core_axis_name="c", subcore_axis_name="s")

    @functools.partial(
        pl.kernel,
        out_type=(jax.ShapeDtypeStruct((NP, D), jnp.float32),
                  jax.ShapeDtypeStruct((NP, D), jnp.float32)),
        mesh=mesh,
        scratch_types=[
            pltpu.VMEM((6, 1, CHUNK), jnp.int32),
            pltpu.VMEM((6, 1, CHUNK), jnp.int32),
            pltpu.VMEM((3, CHUNK, D), jnp.float32),
            pltpu.VMEM_SHARED((NP, D), jnp.float32),
        ] + [pltpu.SemaphoreType.DMA] * 12,
    )
    def agg_kernel(h_hbm, src_hbm, dst_hbm, out0_hbm, out1_hbm,
                   src_r, dst_r, rows, agg_sp, *sems):
        c = lax.axis_index("c")
        s = lax.axis_index("s")
        w = c * NS + s
        gsem = sems[0:3]
        ssem = sems[3:6]
        isem = sems[6:12]

        def fire_idx(v, j):
            pltpu.async_copy(src_hbm.at[w, v], src_r.at[j], isem[j])
            pltpu.async_copy(dst_hbm.at[w, v], dst_r.at[j], isem[j])

        def wait_idx(v, j):
            pltpu.make_async_copy(src_hbm.at[w, v], src_r.at[j],
                                  isem[j]).wait()
            pltpu.make_async_copy(dst_hbm.at[w, v], dst_r.at[j],
                                  isem[j]).wait()

        def fire_gather(j, b):
            pltpu.async_copy(h_hbm.at[src_r.at[j, 0]], rows.at[b], gsem[b])

        def wait_gather(j, b):
            pltpu.make_async_copy(h_hbm.at[src_r.at[j, 0]], rows.at[b],
                                  gsem[b]).wait()

        def fire_scatter(j, b):
            pltpu.async_copy(rows.at[b], agg_sp.at[dst_r.at[j, 0]], ssem[b],
                             add=True)

        def wait_scatter(j, b):
            pltpu.make_async_copy(rows.at[b], agg_sp.at[dst_r.at[j, 0]],
                                  ssem[b]).wait()

        @pl.loop(0, CHUNK)
        def _(i):
            @pl.loop(0, D, step=16)
            def _(j):
                rows[0, i, pl.ds(j, 16)] = jnp.zeros((16,), jnp.float32)

        @pl.loop(0, 5)
        def _(r):
            pltpu.sync_copy(rows.at[0],
                            agg_sp.at[pl.ds(s * RPT + r * CHUNK, CHUNK)])

        pltpu.sync_copy(rows.at[0, pl.ds(0, RPT - 5 * CHUNK)],
                        agg_sp.at[pl.ds(s * RPT + 5 * CHUNK,
                                        RPT - 5 * CHUNK)])

        plsc.subcore_barrier()

        def visit(v, b, j_self, with_scatter_wait):
            b2 = (b + 2) % 3
            j2 = (j_self + 2) % 6
            j5 = (j_self + 5) % 6
            wait_gather(j_self, b)
            fire_scatter(j_self, b)
            if with_scatter_wait:
                wait_scatter((j_self + 5) % 6, b2)
                fire_idx(v + 5, j5)
            wait_idx(v + 2, j2)
            fire_gather(j2, b2)

        pltpu.sync_copy(src_hbm.at[w, 0], src_r.at[0])
        pltpu.sync_copy(dst_hbm.at[w, 0], dst_r.at[0])
        pltpu.sync_copy(src_hbm.at[w, 1], src_r.at[1])
        pltpu.sync_copy(dst_hbm.at[w, 1], dst_r.at[1])
        fire_idx(2, 2)
        fire_idx(3, 3)
        fire_idx(4, 4)
        fire_gather(0, 0)
        fire_gather(1, 1)

        wait_gather(0, 0)
        fire_scatter(0, 0)
        fire_idx(5, 5)
        wait_idx(2, 2)
        fire_gather(2, 2)

        @pl.loop(1, K, step=6)
        def _(i):
            for t in range(6):
                visit(i + t, (1 + t) % 3, (1 + t) % 6, True)

        wait_scatter((K - 1) % 6, (K - 1) % 3)
        wait_gather(K % 6, K % 3)
        wait_gather((K + 1) % 6, (K + 1) % 3)
        wait_idx(K + 2, (K + 2) % 6)
        wait_idx(K + 3, (K + 3) % 6)
        wait_idx(K + 4, (K + 4) % 6)

        plsc.subcore_barrier()

        @pl.when(c == 0)
        def _():
            pltpu.sync_copy(agg_sp.at[pl.ds(s * RPT, RPT)],
                            out0_hbm.at[pl.ds(s * RPT, RPT)])

        @pl.when(c == 1)
        def _():
            pltpu.sync_copy(agg_sp.at[pl.ds(s * RPT, RPT)],
                            out1_hbm.at[pl.ds(s * RPT, RPT)])

    return agg_kernel


_BR = 2048


def _norm(deg):
    return lax.rsqrt(jnp.maximum(deg, 1.0))


def _tc_first(x, deg_out, W):
    def body(x_ref, do_ref, w_ref, o_ref):
        xs = x_ref[...] * _norm(do_ref[...])
        o_ref[...] = jnp.dot(xs, w_ref[...], preferred_element_type=jnp.float32)

    return pl.pallas_call(
        body,
        grid=(NP // _BR,),
        in_specs=[
            pl.BlockSpec((_BR, 128), lambda i: (i, 0)),
            pl.BlockSpec((_BR, 1), lambda i: (i, 0)),
            pl.BlockSpec((128, 128), lambda i: (0, 0)),
        ],
        out_specs=pl.BlockSpec((_BR, 128), lambda i: (i, 0)),
        out_shape=jax.ShapeDtypeStruct((NP, 128), jnp.float32),
    )(x, deg_out, W)


def _tc_mid(a0, a1, deg_in, deg_out, b, W, DO):
    def body(a0_ref, a1_ref, di_ref, do_ref, b_ref, w_ref, o_ref):
        a = (a0_ref[...] + a1_ref[...]) * _norm(di_ref[...]) + b_ref[...]
        xs = jax.nn.relu(a) * _norm(do_ref[...])
        o_ref[...] = jnp.dot(xs, w_ref[...], preferred_element_type=jnp.float32)

    return pl.pallas_call(
        body,
        grid=(NP // _BR,),
        in_specs=[
            pl.BlockSpec((_BR, 128), lambda i: (i, 0)),
            pl.BlockSpec((_BR, 128), lambda i: (i, 0)),
            pl.BlockSpec((_BR, 1), lambda i: (i, 0)),
            pl.BlockSpec((_BR, 1), lambda i: (i, 0)),
            pl.BlockSpec((1, 128), lambda i: (0, 0)),
            pl.BlockSpec((128, DO), lambda i: (0, 0)),
        ],
        out_specs=pl.BlockSpec((_BR, DO), lambda i: (i, 0)),
        out_shape=jax.ShapeDtypeStruct((NP, DO), jnp.float32),
    )(a0, a1, deg_in, deg_out, b, W)


def _tc_final(a0, a1, deg_in, b):
    def body(a0_ref, a1_ref, di_ref, b_ref, o_ref):
        a = (a0_ref[...] + a1_ref[...]) * _norm(di_ref[...]) + b_ref[...]
        o_ref[...] = jax.nn.relu(a[:, :40])

    return pl.pallas_call(
        body,
        grid=(10,),
        in_specs=[
            pl.BlockSpec((1000, 128), lambda i: (i, 0)),
            pl.BlockSpec((1000, 128), lambda i: (i, 0)),
            pl.BlockSpec((1000, 1), lambda i: (i, 0)),
            pl.BlockSpec((1, 128), lambda i: (0, 0)),
        ],
        out_specs=pl.BlockSpec((1000, 40), lambda i: (i, 0)),
        out_shape=jax.ShapeDtypeStruct((N_NODES, 40), jnp.float32),
    )(a0, a1, deg_in, b)


def kernel(in_feat, edge_index, W1, b1, W2, b2, W3, b3):
    src = edge_index[0].astype(jnp.int32)
    dst = edge_index[1].astype(jnp.int32)
    pad_idx = PAD + jnp.arange(EP - E, dtype=jnp.int32) % (NP - N_NODES)
    src_flat = jnp.concatenate([src, pad_idx])
    dst_flat = jnp.concatenate([dst, pad_idx])
    src_d = src_flat.reshape(NS, KD, DCHUNK)
    dst_d = dst_flat.reshape(NS, KD, DCHUNK)
    dummy = (PAD + jnp.arange(32 * (KX - K) * CHUNK, dtype=jnp.int32)
             % (NP - N_NODES)).reshape(32, KX - K, CHUNK)
    src_p = jnp.concatenate([src_flat.reshape(32, K, CHUNK), dummy],
                            axis=1).reshape(32, KX, 1, CHUNK)
    dst_p = jnp.concatenate([dst_flat.reshape(32, K, CHUNK), dummy],
                            axis=1).reshape(32, KX, 1, CHUNK)

    x_p = jnp.concatenate(
        [in_feat, jnp.zeros((NP - N_NODES, 128), jnp.float32)])

    deg_out, deg_in = _sc_degrees()(src_d, dst_d)
    deg_out = deg_out.reshape(NP, 1)
    deg_in = deg_in.reshape(NP, 1)

    W3p = jnp.pad(W3, ((0, 0), (0, 128 - 40)))
    b3p = jnp.pad(b3, (0, 128 - 40)).reshape(1, 128)

    h1 = _tc_first(x_p, deg_out, W1)
    a1_0, a1_1 = _sc_aggregate(128)(h1, src_p, dst_p)
    h2 = _tc_mid(a1_0, a1_1, deg_in, deg_out, b1.reshape(1, 128), W2, 128)
    a2_0, a2_1 = _sc_aggregate(128)(h2, src_p, dst_p)
    h3 = _tc_mid(a2_0, a2_1, deg_in, deg_out, b2.reshape(1, 128), W3p, 128)
    a3_0, a3_1 = _sc_aggregate(128)(h3, src_p, dst_p)
    return _tc_final(a3_0, a3_1, deg_in, b3p)
